# baseline (speedup 1.0000x reference)
import jax
import jax.numpy as jnp
from jax.experimental import pallas as pl
from jax.experimental.pallas import tpu as pltpu


_TOKENS_PER_STEP = 8192
_ROWS_PER_ITER = 32


def _round_up(x, m):
    return (x + m - 1) // m * m


def _make_gather_body(tokens_per_step, s, d2):
    def _body(ids_ref, tbl_ref, out_ref):
        tok0 = pl.program_id(0) * tokens_per_step

        def chunk(c, carry):
            base = tok0 + c * s
            rows = []
            for u in range(s):
                rows.append(tbl_ref[ids_ref[base + u]])
            for g in range(s // 8):
                out_ref[c, pl.ds(pl.multiple_of(8 * g, 8), 8), :] = (
                    jnp.concatenate(rows[8 * g:8 * g + 8], axis=0))
            return carry

        jax.lax.fori_loop(0, tokens_per_step // s, chunk, 0)

    return _body


@jax.jit
def kernel(embedding, feature_embed, item_feature, inputs, A, combined):
    del feature_embed, item_feature, A
    n_node = embedding.shape[0]
    n_pad, d2 = combined.shape
    b, s = inputs.shape
    n = b * s
    h2 = 2 * d2

    ids = jnp.clip(inputs.reshape(-1).astype(jnp.int32), 0, n_node - 1)
    tokens_per_step = min(_TOKENS_PER_STEP, n)
    grid = (n // tokens_per_step,)

    tbl = combined.reshape(n_pad, 1, d2)

    out = pl.pallas_call(
        _make_gather_body(tokens_per_step, s, d2),
        out_shape=jax.ShapeDtypeStruct((b, s, d2), jnp.float32),
        grid=grid,
        in_specs=[
            pl.BlockSpec(memory_space=pltpu.SMEM),
            pl.BlockSpec((n_pad, 1, d2), lambda i: (0, 0, 0)),
        ],
        out_specs=pl.BlockSpec((tokens_per_step // s, s, d2),
                               lambda i: (i, 0, 0)),
        compiler_params=pltpu.CompilerParams(
            dimension_semantics=("parallel",),
            vmem_limit_bytes=int(56 << 20),
        ),
    )(ids, tbl)

    return out


# speedup vs baseline: 1.0288x; 1.0288x over previous
import jax
import jax.numpy as jnp
from jax.experimental import pallas as pl
from jax.experimental.pallas import tpu as pltpu


_TOKENS_PER_STEP = 8192
_ROWS_PER_ITER = 32


def _round_up(x, m):
    return (x + m - 1) // m * m


def _make_gather_body(tokens_per_step, d2):
    def _body(ids_ref, tbl2_ref, out_ref):
        tok0 = pl.program_id(0) * tokens_per_step
        tbl_ref = tbl2_ref.reshape(tbl2_ref.shape[0], 1, d2)

        def chunk(c, carry):
            base = tok0 + c * _ROWS_PER_ITER
            row0 = c * _ROWS_PER_ITER
            rows = []
            for u in range(_ROWS_PER_ITER):
                rows.append(tbl_ref[ids_ref[base + u]])
            for g in range(_ROWS_PER_ITER // 8):
                out_ref[pl.ds(pl.multiple_of(row0 + 8 * g, 8), 8), :] = (
                    jnp.concatenate(rows[8 * g:8 * g + 8], axis=0))
            return carry

        jax.lax.fori_loop(0, tokens_per_step // _ROWS_PER_ITER, chunk, 0)

    return _body


@jax.jit
def kernel(embedding, feature_embed, item_feature, inputs, A, combined):
    del feature_embed, item_feature, A
    n_node = embedding.shape[0]
    n_pad, d2 = combined.shape
    b, s = inputs.shape
    n = b * s
    h2 = 2 * d2

    ids = jnp.clip(inputs.reshape(-1).astype(jnp.int32), 0, n_node - 1)
    np_tok = _round_up(n, _TOKENS_PER_STEP)
    if np_tok != n:
        ids = jnp.pad(ids, (0, np_tok - n))
    grid = (np_tok // _TOKENS_PER_STEP,)

    tbl = combined

    out = pl.pallas_call(
        _make_gather_body(_TOKENS_PER_STEP, d2),
        out_shape=jax.ShapeDtypeStruct((np_tok, d2), jnp.float32),
        grid=grid,
        in_specs=[
            pl.BlockSpec(memory_space=pltpu.SMEM),
            pl.BlockSpec((n_pad, d2), lambda i: (0, 0)),
        ],
        out_specs=pl.BlockSpec((_TOKENS_PER_STEP, d2), lambda i: (i, 0)),
        compiler_params=pltpu.CompilerParams(
            dimension_semantics=("parallel",),
            vmem_limit_bytes=int(56 << 20),
        ),
    )(ids, tbl)

    return out[:n].reshape(b, s, d2)


# speedup vs baseline: 1.1038x; 1.0729x over previous
import jax
import jax.numpy as jnp
from jax.experimental import pallas as pl
from jax.experimental.pallas import tpu as pltpu


_TOKENS_PER_STEP = 16384
_ROWS_PER_ITER = 64


def _round_up(x, m):
    return (x + m - 1) // m * m


def _make_gather_body(tokens_per_step, d2):
    def _body(ids_ref, tbl_ref, out_ref):
        tok0 = pl.program_id(0) * tokens_per_step

        def chunk(c, carry):
            base = tok0 + c * _ROWS_PER_ITER
            row0 = c * _ROWS_PER_ITER
            rows = []
            for u in range(_ROWS_PER_ITER):
                rows.append(tbl_ref[ids_ref[base + u]])
            for g in range(_ROWS_PER_ITER // 8):
                out_ref[pl.ds(pl.multiple_of(row0 + 8 * g, 8), 8), :] = (
                    jnp.concatenate(rows[8 * g:8 * g + 8], axis=0))
            return carry

        jax.lax.fori_loop(0, tokens_per_step // _ROWS_PER_ITER, chunk, 0)

    return _body


@jax.jit
def kernel(embedding, feature_embed, item_feature, inputs, A, combined):
    del feature_embed, item_feature, A
    n_node = embedding.shape[0]
    n_pad, d2 = combined.shape
    b, s = inputs.shape
    n = b * s
    h2 = 2 * d2

    ids = jnp.clip(inputs.reshape(-1).astype(jnp.int32), 0, n_node - 1)
    np_tok = _round_up(n, _TOKENS_PER_STEP)
    if np_tok != n:
        ids = jnp.pad(ids, (0, np_tok - n))
    grid = (np_tok // _TOKENS_PER_STEP,)

    tbl = combined.reshape(n_pad, 1, d2)

    out = pl.pallas_call(
        _make_gather_body(_TOKENS_PER_STEP, d2),
        out_shape=jax.ShapeDtypeStruct((np_tok, d2), jnp.float32),
        grid=grid,
        in_specs=[
            pl.BlockSpec(memory_space=pltpu.SMEM),
            pl.BlockSpec((n_pad, 1, d2), lambda i: (0, 0, 0)),
        ],
        out_specs=pl.BlockSpec((_TOKENS_PER_STEP, d2), lambda i: (i, 0)),
        compiler_params=pltpu.CompilerParams(
            dimension_semantics=("parallel",),
            vmem_limit_bytes=int(56 << 20),
        ),
    )(ids, tbl)

    return out[:n].reshape(b, s, d2)


# speedup vs baseline: 1.1359x; 1.0292x over previous
import jax
import jax.numpy as jnp
from jax.experimental import pallas as pl
from jax.experimental.pallas import tpu as pltpu


_TOKENS_PER_STEP = 16384
_ROWS_PER_ITER = 128


def _round_up(x, m):
    return (x + m - 1) // m * m


def _make_gather_body(tokens_per_step, d2):
    def _body(ids_ref, tbl_ref, out_ref):
        tok0 = pl.program_id(0) * tokens_per_step

        def chunk(c, carry):
            base = tok0 + c * _ROWS_PER_ITER
            row0 = c * _ROWS_PER_ITER
            rows = []
            for u in range(_ROWS_PER_ITER):
                rows.append(tbl_ref[ids_ref[base + u]])
            for g in range(_ROWS_PER_ITER // 8):
                out_ref[pl.ds(pl.multiple_of(row0 + 8 * g, 8), 8), :] = (
                    jnp.concatenate(rows[8 * g:8 * g + 8], axis=0))
            return carry

        jax.lax.fori_loop(0, tokens_per_step // _ROWS_PER_ITER, chunk, 0)

    return _body


@jax.jit
def kernel(embedding, feature_embed, item_feature, inputs, A, combined):
    del feature_embed, item_feature, A
    n_node = embedding.shape[0]
    n_pad, d2 = combined.shape
    b, s = inputs.shape
    n = b * s
    h2 = 2 * d2

    ids = jnp.clip(inputs.reshape(-1).astype(jnp.int32), 0, n_node - 1)
    np_tok = _round_up(n, _TOKENS_PER_STEP)
    if np_tok != n:
        ids = jnp.pad(ids, (0, np_tok - n))
    grid = (np_tok // _TOKENS_PER_STEP,)

    tbl = combined.reshape(n_pad, 1, d2)

    out = pl.pallas_call(
        _make_gather_body(_TOKENS_PER_STEP, d2),
        out_shape=jax.ShapeDtypeStruct((np_tok, d2), jnp.float32),
        grid=grid,
        in_specs=[
            pl.BlockSpec(memory_space=pltpu.SMEM),
            pl.BlockSpec((n_pad, 1, d2), lambda i: (0, 0, 0)),
        ],
        out_specs=pl.BlockSpec((_TOKENS_PER_STEP, d2), lambda i: (i, 0)),
        compiler_params=pltpu.CompilerParams(
            dimension_semantics=("parallel",),
            vmem_limit_bytes=int(56 << 20),
        ),
    )(ids, tbl)

    return out[:n].reshape(b, s, d2)


# speedup vs baseline: 1.1516x; 1.0137x over previous
import jax
import jax.numpy as jnp
from jax.experimental import pallas as pl
from jax.experimental.pallas import tpu as pltpu


_TOKENS_PER_STEP = 16384
_ROWS_PER_ITER = 256


def _round_up(x, m):
    return (x + m - 1) // m * m


def _make_gather_body(tokens_per_step, d2):
    def _body(ids_ref, tbl_ref, out_ref):
        tok0 = pl.program_id(0) * tokens_per_step

        def chunk(c, carry):
            base = tok0 + c * _ROWS_PER_ITER
            row0 = c * _ROWS_PER_ITER
            rows = []
            for u in range(_ROWS_PER_ITER):
                rows.append(tbl_ref[ids_ref[base + u]])
            for g in range(_ROWS_PER_ITER // 8):
                out_ref[pl.ds(pl.multiple_of(row0 + 8 * g, 8), 8), :] = (
                    jnp.concatenate(rows[8 * g:8 * g + 8], axis=0))
            return carry

        jax.lax.fori_loop(0, tokens_per_step // _ROWS_PER_ITER, chunk, 0)

    return _body


@jax.jit
def kernel(embedding, feature_embed, item_feature, inputs, A, combined):
    del feature_embed, item_feature, A
    n_node = embedding.shape[0]
    n_pad, d2 = combined.shape
    b, s = inputs.shape
    n = b * s
    h2 = 2 * d2

    ids = jnp.clip(inputs.reshape(-1).astype(jnp.int32), 0, n_node - 1)
    np_tok = _round_up(n, _TOKENS_PER_STEP)
    if np_tok != n:
        ids = jnp.pad(ids, (0, np_tok - n))
    grid = (np_tok // _TOKENS_PER_STEP,)

    tbl = combined.reshape(n_pad, 1, d2)

    out = pl.pallas_call(
        _make_gather_body(_TOKENS_PER_STEP, d2),
        out_shape=jax.ShapeDtypeStruct((np_tok, d2), jnp.float32),
        grid=grid,
        in_specs=[
            pl.BlockSpec(memory_space=pltpu.SMEM),
            pl.BlockSpec((n_pad, 1, d2), lambda i: (0, 0, 0)),
        ],
        out_specs=pl.BlockSpec((_TOKENS_PER_STEP, d2), lambda i: (i, 0)),
        compiler_params=pltpu.CompilerParams(
            dimension_semantics=("parallel",),
            vmem_limit_bytes=int(56 << 20),
        ),
    )(ids, tbl)

    return out[:n].reshape(b, s, d2)
